# baseline (speedup 1.0000x reference)
import functools

import jax
import jax.numpy as jnp
from jax import lax
from jax.experimental import pallas as pl
from jax.experimental.pallas import tpu as pltpu
from jax.experimental.pallas import tpu_sc as plsc

BATCH = 16384
DIM = 64
BITS = 18
SHIFT = 32 - BITS
HASH_A_USER = 2654435761
HASH_A_ITEM = 2246822519

NC = 2
NS = 16
NW = NC * NS
B_PER_W = BATCH // NW
N_CHUNK = 4
CHUNK = B_PER_W // N_CHUNK
L = 16


def _sc_gather_body(user_hbm, item_hbm, utab_hbm, itab_hbm,
                    uout_hbm, iout_hbm,
                    raw_u, raw_i, uidx, iidx, urows, irows, sem):
    wid = lax.axis_index("s") * NC + lax.axis_index("c")
    base = wid * B_PER_W

    pltpu.sync_copy(user_hbm.at[pl.ds(base, B_PER_W)], raw_u)
    pltpu.sync_copy(item_hbm.at[pl.ds(base, B_PER_W)], raw_i)

    au = jnp.uint32(HASH_A_USER)
    ai = jnp.uint32(HASH_A_ITEM)
    sh = jnp.uint32(SHIFT)
    for k in range(B_PER_W // L):
        r = k // (CHUNK // L)
        c = (k % (CHUNK // L)) * L
        u = raw_u[pl.ds(k * L, L)].astype(jnp.uint32)
        i = raw_i[pl.ds(k * L, L)].astype(jnp.uint32)
        uidx[r, pl.ds(c, L)] = ((u * au) >> sh).astype(jnp.int32)
        iidx[r, pl.ds(c, L)] = ((i * ai) >> sh).astype(jnp.int32)

    copies = []
    for j in range(N_CHUNK):
        copies.append(pltpu.async_copy(utab_hbm.at[uidx.at[j]], urows.at[j], sem))
        copies.append(pltpu.async_copy(itab_hbm.at[iidx.at[j]], irows.at[j], sem))
    for cp in copies:
        cp.wait()

    for j in range(N_CHUNK):
        pltpu.sync_copy(urows.at[j], uout_hbm.at[pl.ds(base + j * CHUNK, CHUNK)])
        pltpu.sync_copy(irows.at[j], iout_hbm.at[pl.ds(base + j * CHUNK, CHUNK)])


_sc_gather = functools.partial(
    pl.kernel,
    out_type=(
        jax.ShapeDtypeStruct((BATCH, DIM), jnp.float32),
        jax.ShapeDtypeStruct((BATCH, DIM), jnp.float32),
    ),
    mesh=plsc.VectorSubcoreMesh(core_axis_name="c", subcore_axis_name="s"),
    scratch_types=[
        pltpu.VMEM((B_PER_W,), jnp.int32),
        pltpu.VMEM((B_PER_W,), jnp.int32),
        pltpu.VMEM((N_CHUNK, CHUNK), jnp.int32),
        pltpu.VMEM((N_CHUNK, CHUNK), jnp.int32),
        pltpu.VMEM((N_CHUNK, CHUNK, DIM), jnp.float32),
        pltpu.VMEM((N_CHUNK, CHUNK, DIM), jnp.float32),
        pltpu.SemaphoreType.DMA,
    ],
    compiler_params=pltpu.CompilerParams(use_tc_tiling_on_sc=False),
)(_sc_gather_body)


BLK = 2048


def _mlp_body(u_ref, v_ref, w1_ref, b1_ref, w2_ref, b2_ref, o_ref):
    x = u_ref[...] * v_ref[...]
    h = jnp.dot(x, w1_ref[...], preferred_element_type=jnp.float32) + b1_ref[...]
    h = jnp.maximum(h, 0.0)
    z = jnp.dot(h, w2_ref[...], preferred_element_type=jnp.float32) + b2_ref[0, 0]
    o_ref[...] = 1.0 / (1.0 + jnp.exp(-z))


def _mlp(u_emb, i_emb, W1, b1, W2, b2):
    grid = (BATCH // BLK,)
    return pl.pallas_call(
        _mlp_body,
        grid=grid,
        in_specs=[
            pl.BlockSpec((BLK, DIM), lambda i: (i, 0)),
            pl.BlockSpec((BLK, DIM), lambda i: (i, 0)),
            pl.BlockSpec((DIM, 20), lambda i: (0, 0)),
            pl.BlockSpec((1, 20), lambda i: (0, 0)),
            pl.BlockSpec((20, 1), lambda i: (0, 0)),
            pl.BlockSpec((1, 1), lambda i: (0, 0)),
        ],
        out_specs=pl.BlockSpec((BLK, 1), lambda i: (i, 0)),
        out_shape=jax.ShapeDtypeStruct((BATCH, 1), jnp.float32),
    )(u_emb, i_emb, W1, b1, W2, b2)


def kernel(user, item, user_table, item_table, W1, b1, W2, b2):
    u_emb, i_emb = _sc_gather(user, item, user_table, item_table)
    out = _mlp(u_emb, i_emb, W1,
               b1.reshape(1, 20), W2, b2.reshape(1, 1))
    return out.reshape(-1)
